# baseline (speedup 1.0000x reference)
import functools

import jax
import jax.numpy as jnp
from jax import lax
from jax.experimental import pallas as pl
from jax.experimental.pallas import tpu as pltpu
from jax.experimental.pallas import tpu_sc as plsc

NUM_VOCAB = 1000000
NUM_EMBED = 64
NUM_TOKEN = 200
BATCH = 1024
TOK_PAD = 256

NW = 32
B_PER_W = BATCH // NW
B_TOTAL = BATCH * NUM_TOKEN
CHUNK_A = 128
CHUNK_B = NUM_TOKEN - CHUNK_A
LANES = 16
C_PER_ROW = NUM_EMBED // LANES


def _tok_split_kernel(t_ref, o_ref):
  x = t_ref[...]
  y = jnp.stack([x[:, :128], x[:, 128:]], axis=1)
  o_ref[...] = y.reshape(o_ref.shape)


@jax.jit
def _tok_split(tokens_p):
  return pl.pallas_call(
      _tok_split_kernel,
      grid=(16,),
      in_specs=[pl.BlockSpec((64, TOK_PAD), lambda i: (i, 0))],
      out_specs=pl.BlockSpec((128, 128), lambda i: (i, 0)),
      out_shape=jax.ShapeDtypeStruct((2 * BATCH, 128), jnp.int32),
  )(tokens_p)


def _emb_kernel(tok_hbm, table_hbm, pos_hbm, out_hbm, idx_v, pos_v,
                gbuf0, gbuf1, rows0, rows1, gsem0, gsem1, ssem0, ssem1):
  wid = lax.axis_index("s") * 2 + lax.axis_index("c")
  row0 = wid * 2 * B_PER_W
  base = wid * B_PER_W * NUM_TOKEN

  pltpu.sync_copy(tok_hbm.at[pl.ds(row0, 2 * B_PER_W)], idx_v)
  pltpu.sync_copy(pos_hbm, pos_v)

  gbufs = (gbuf0, gbuf1)
  rows = (rows0, rows1)
  gsems = (gsem0, gsem1)
  ssems = (ssem0, ssem1)

  def gather(q, g):
    pltpu.async_copy(table_hbm.at[idx_v.at[q]], gbufs[g], gsems[g])

  def gwait(g):
    pltpu.make_async_copy(table_hbm.at[idx_v.at[0]], gbufs[g],
                          gsems[g]).wait()

  gather(0, 0)

  def batch_body(p, carry):
    for h, phase, nrow in ((0, 0, CHUNK_A), (1, CHUNK_A, CHUNK_B)):
      q = 2 * p + h
      gwait(h)

      @pl.when(q + 1 < 2 * B_PER_W)
      def _():
        gather(q + 1, 1 - h)

      @pl.when(p >= 1)
      def _():
        pltpu.make_async_copy(rows[h].at[pl.ds(0, nrow)],
                              out_hbm.at[pl.ds(0, nrow)], ssems[h]).wait()

      def row_body(r, c2):
        for c in range(C_PER_ROW):
          sl = pl.ds(c * LANES, LANES)
          rows[h][r, sl] = gbufs[h][r, sl] + pos_v[phase + r, sl]
        return c2

      lax.fori_loop(0, nrow, row_body, 0)
      pltpu.async_copy(
          rows[h].at[pl.ds(0, nrow)],
          out_hbm.at[pl.ds(base + p * NUM_TOKEN + phase, nrow)], ssems[h])
    return carry

  lax.fori_loop(0, B_PER_W, batch_body, 0)
  for h, nrow in ((0, CHUNK_A), (1, CHUNK_B)):
    pltpu.make_async_copy(rows[h].at[pl.ds(0, nrow)],
                          out_hbm.at[pl.ds(0, nrow)], ssems[h]).wait()


@jax.jit
def _emb(tok2, table, positionembed):
  mesh = plsc.VectorSubcoreMesh(core_axis_name="c", subcore_axis_name="s")
  run = functools.partial(
      pl.kernel,
      mesh=mesh,
      compiler_params=pltpu.CompilerParams(use_tc_tiling_on_sc=False,
                                           needs_layout_passes=False),
      out_type=jax.ShapeDtypeStruct((B_TOTAL, NUM_EMBED), jnp.float32),
      scratch_types=[
          pltpu.VMEM((2 * B_PER_W, 128), jnp.int32),
          pltpu.VMEM((NUM_TOKEN, NUM_EMBED), jnp.float32),
          pltpu.VMEM((CHUNK_A, NUM_EMBED), jnp.float32),
          pltpu.VMEM((CHUNK_A, NUM_EMBED), jnp.float32),
          pltpu.VMEM((CHUNK_A, NUM_EMBED), jnp.float32),
          pltpu.VMEM((CHUNK_A, NUM_EMBED), jnp.float32),
          pltpu.SemaphoreType.DMA,
          pltpu.SemaphoreType.DMA,
          pltpu.SemaphoreType.DMA,
          pltpu.SemaphoreType.DMA,
      ],
  )(_emb_kernel)
  return run(tok2, table, positionembed)


def kernel(tokens, table, positionembed):
  tokens_p = jnp.pad(tokens.astype(jnp.int32),
                     ((0, 0), (0, TOK_PAD - NUM_TOKEN)))
  tok2 = _tok_split(tokens_p)
  out = _emb(tok2, table, positionembed)
  return out.reshape(BATCH, NUM_TOKEN, NUM_EMBED)


# speedup vs baseline: 2.3405x; 2.3405x over previous
import functools

import jax
import jax.numpy as jnp
from jax import lax
from jax.experimental import pallas as pl
from jax.experimental.pallas import tpu as pltpu
from jax.experimental.pallas import tpu_sc as plsc

NUM_VOCAB = 1000000
NUM_EMBED = 64
NUM_TOKEN = 200
BATCH = 1024
TOK_PAD = 256

NW = 32
B_PER_W = BATCH // NW
B_TOTAL = BATCH * NUM_TOKEN
CHUNK_A = 128
CHUNK_B = NUM_TOKEN - CHUNK_A
LANES = 16
C_PER_ROW = NUM_EMBED // LANES


def _tok_split_kernel(t_ref, o_ref):
  x = t_ref[...].astype(jnp.float32)
  y = jnp.stack([x[:, :64], x[:, 64:128], x[:, 128:192], x[:, 192:]], axis=1)
  o_ref[...] = y.reshape(o_ref.shape)


@jax.jit
def _tok_split(tokens_p):
  return pl.pallas_call(
      _tok_split_kernel,
      grid=(16,),
      in_specs=[pl.BlockSpec((64, TOK_PAD), lambda i: (i, 0))],
      out_specs=pl.BlockSpec((256, 64), lambda i: (i, 0)),
      out_shape=jax.ShapeDtypeStruct((4 * BATCH, 64), jnp.float32),
  )(tokens_p)


def _emb_kernel(tok_hbm, table_hbm, pos_hbm, out_hbm, idxf_v, idx_v, pos_v,
                bufa, bufb, gsem):
  wid = lax.axis_index("s") * 2 + lax.axis_index("c")
  base = wid * B_PER_W * NUM_TOKEN

  pltpu.sync_copy(tok_hbm.at[pl.ds(wid * 4 * B_PER_W, 4 * B_PER_W)], idxf_v)
  pltpu.sync_copy(pos_hbm, pos_v)

  def conv_body(p, carry):
    fb = 4 * p
    ob = p * NUM_TOKEN
    for r, off in ((0, 0), (0, 16), (0, 32), (0, 48),
                   (1, 0), (1, 16), (1, 32), (1, 48),
                   (2, 0), (2, 16), (2, 32), (2, 48),
                   (3, 0)):
      v = idxf_v[fb + r, pl.ds(off, LANES)].astype(jnp.int32)
      dst = pl.multiple_of(ob + 64 * r + off, 8)
      idx_v[pl.ds(dst, LANES)] = v
    return carry

  lax.fori_loop(0, B_PER_W, conv_body, 0)

  def batch_body(p, carry):
    ob = p * NUM_TOKEN
    cpa = pltpu.async_copy(table_hbm.at[idx_v.at[pl.ds(ob, CHUNK_A)]],
                           bufa, gsem)
    cpb = pltpu.async_copy(table_hbm.at[idx_v.at[pl.ds(ob + CHUNK_A,
                                                       CHUNK_B)]],
                           bufb, gsem)
    cpa.wait()
    cpb.wait()

    def row_a(r, c2):
      for c in range(C_PER_ROW):
        sl = pl.ds(c * LANES, LANES)
        bufa[r, sl] = bufa[r, sl] + pos_v[r, sl]
      return c2

    def row_b(r, c2):
      for c in range(C_PER_ROW):
        sl = pl.ds(c * LANES, LANES)
        bufb[r, sl] = bufb[r, sl] + pos_v[CHUNK_A + r, sl]
      return c2

    lax.fori_loop(0, CHUNK_A, row_a, 0)
    lax.fori_loop(0, CHUNK_B, row_b, 0)
    pltpu.sync_copy(bufa, out_hbm.at[pl.ds(base + ob, CHUNK_A)])
    pltpu.sync_copy(bufb, out_hbm.at[pl.ds(base + ob + CHUNK_A, CHUNK_B)])
    return carry

  lax.fori_loop(0, B_PER_W, batch_body, 0)


@jax.jit
def _emb(tok4, table, positionembed):
  mesh = plsc.VectorSubcoreMesh(core_axis_name="c", subcore_axis_name="s")
  run = functools.partial(
      pl.kernel,
      mesh=mesh,
      compiler_params=pltpu.CompilerParams(use_tc_tiling_on_sc=False,
                                           needs_layout_passes=False),
      out_type=jax.ShapeDtypeStruct((B_TOTAL, NUM_EMBED), jnp.float32),
      scratch_types=[
          pltpu.VMEM((4 * B_PER_W, NUM_EMBED), jnp.float32),
          pltpu.VMEM((B_PER_W * NUM_TOKEN + 8, ), jnp.int32),
          pltpu.VMEM((NUM_TOKEN, NUM_EMBED), jnp.float32),
          pltpu.VMEM((CHUNK_A, NUM_EMBED), jnp.float32),
          pltpu.VMEM((CHUNK_B, NUM_EMBED), jnp.float32),
          pltpu.SemaphoreType.DMA,
      ],
  )(_emb_kernel)
  return run(tok4, table, positionembed)


def kernel(tokens, table, positionembed):
  tokens_p = jnp.pad(tokens.astype(jnp.int32),
                     ((0, 0), (0, TOK_PAD - NUM_TOKEN)))
  tok4 = _tok_split(tokens_p)
  out = _emb(tok4, table, positionembed)
  return out.reshape(BATCH, NUM_TOKEN, NUM_EMBED)


# speedup vs baseline: 2.3459x; 1.0023x over previous
import functools

import jax
import jax.numpy as jnp
from jax import lax
from jax.experimental import pallas as pl
from jax.experimental.pallas import tpu as pltpu
from jax.experimental.pallas import tpu_sc as plsc

NUM_VOCAB = 1000000
NUM_EMBED = 64
NUM_TOKEN = 200
BATCH = 1024
TOK_PAD = 256

NW = 32
B_PER_W = BATCH // NW
B_TOTAL = BATCH * NUM_TOKEN
CHUNK_A = 128
CHUNK_B = NUM_TOKEN - CHUNK_A
LANES = 16
C_PER_ROW = NUM_EMBED // LANES


def _tok_split_kernel(t_ref, o_ref):
  o_ref[...] = t_ref[...].reshape(o_ref.shape)


@jax.jit
def _tok_split(tokens_p):
  return pl.pallas_call(
      _tok_split_kernel,
      grid=(16,),
      in_specs=[pl.BlockSpec((64, TOK_PAD), lambda i: (i, 0))],
      out_specs=pl.BlockSpec((64 * TOK_PAD,), lambda i: (i,)),
      out_shape=jax.ShapeDtypeStruct((BATCH * TOK_PAD,), jnp.int32),
  )(tokens_p)


def _emb_kernel(tok_hbm, table_hbm, pos_hbm, out_hbm, idx_v, pos_v,
                bufa, bufb, gsem):
  wid = lax.axis_index("s") * 2 + lax.axis_index("c")
  base = wid * B_PER_W * NUM_TOKEN

  pltpu.sync_copy(tok_hbm.at[pl.ds(wid * B_PER_W * TOK_PAD,
                                   B_PER_W * TOK_PAD)], idx_v)
  pltpu.sync_copy(pos_hbm, pos_v)

  def batch_body(p, carry):
    ob = p * NUM_TOKEN
    ib = p * TOK_PAD
    cpa = pltpu.async_copy(table_hbm.at[idx_v.at[pl.ds(ib, CHUNK_A)]],
                           bufa, gsem)
    cpb = pltpu.async_copy(table_hbm.at[idx_v.at[pl.ds(ib + CHUNK_A,
                                                       CHUNK_B)]],
                           bufb, gsem)
    cpa.wait()
    cpb.wait()

    def row_a(r, c2):
      for c in range(C_PER_ROW):
        sl = pl.ds(c * LANES, LANES)
        bufa[r, sl] = bufa[r, sl] + pos_v[r, sl]
      return c2

    def row_b(r, c2):
      for c in range(C_PER_ROW):
        sl = pl.ds(c * LANES, LANES)
        bufb[r, sl] = bufb[r, sl] + pos_v[CHUNK_A + r, sl]
      return c2

    lax.fori_loop(0, CHUNK_A, row_a, 0)
    lax.fori_loop(0, CHUNK_B, row_b, 0)
    pltpu.sync_copy(bufa, out_hbm.at[pl.ds(base + ob, CHUNK_A)])
    pltpu.sync_copy(bufb, out_hbm.at[pl.ds(base + ob + CHUNK_A, CHUNK_B)])
    return carry

  lax.fori_loop(0, B_PER_W, batch_body, 0)


@jax.jit
def _emb(tok1, table, positionembed):
  mesh = plsc.VectorSubcoreMesh(core_axis_name="c", subcore_axis_name="s")
  run = functools.partial(
      pl.kernel,
      mesh=mesh,
      compiler_params=pltpu.CompilerParams(use_tc_tiling_on_sc=False,
                                           needs_layout_passes=False),
      out_type=jax.ShapeDtypeStruct((B_TOTAL, NUM_EMBED), jnp.float32),
      scratch_types=[
          pltpu.VMEM((B_PER_W * TOK_PAD,), jnp.int32),
          pltpu.VMEM((NUM_TOKEN, NUM_EMBED), jnp.float32),
          pltpu.VMEM((CHUNK_A, NUM_EMBED), jnp.float32),
          pltpu.VMEM((CHUNK_B, NUM_EMBED), jnp.float32),
          pltpu.SemaphoreType.DMA,
      ],
  )(_emb_kernel)
  return run(tok1, table, positionembed)


def kernel(tokens, table, positionembed):
  tokens_p = jnp.pad(tokens.astype(jnp.int32),
                     ((0, 0), (0, TOK_PAD - NUM_TOKEN)))
  tok1 = _tok_split(tokens_p)
  out = _emb(tok1, table, positionembed)
  return out.reshape(BATCH, NUM_TOKEN, NUM_EMBED)


# speedup vs baseline: 2.4631x; 1.0499x over previous
import functools

import jax
import jax.numpy as jnp
from jax import lax
from jax.experimental import pallas as pl
from jax.experimental.pallas import tpu as pltpu
from jax.experimental.pallas import tpu_sc as plsc

NUM_VOCAB = 1000000
NUM_EMBED = 64
NUM_TOKEN = 200
BATCH = 1024
TOK_PAD = 256

NW = 32
B_PER_W = BATCH // NW
B_TOTAL = BATCH * NUM_TOKEN
CHUNK_A = 128
CHUNK_B = NUM_TOKEN - CHUNK_A
LANES = 16
C_PER_ROW = NUM_EMBED // LANES


def _tok_split_kernel(t_ref, o_ref):
  o_ref[...] = t_ref[...].reshape(o_ref.shape)


@jax.jit
def _tok_split(tokens_p):
  return pl.pallas_call(
      _tok_split_kernel,
      grid=(16,),
      in_specs=[pl.BlockSpec((64, TOK_PAD), lambda i: (i, 0))],
      out_specs=pl.BlockSpec((64 * TOK_PAD,), lambda i: (i,)),
      out_shape=jax.ShapeDtypeStruct((BATCH * TOK_PAD,), jnp.int32),
  )(tokens_p)


def _emb_kernel(tok_hbm, table_hbm, pos_hbm, out_hbm, idx_v, pos_v,
                bufa, bufb, bufa2, bufb2, gsem, gsem2, ssem, ssem2):
  wid = lax.axis_index("s") * 2 + lax.axis_index("c")
  base = wid * B_PER_W * NUM_TOKEN

  pltpu.sync_copy(tok_hbm.at[pl.ds(wid * B_PER_W * TOK_PAD,
                                   B_PER_W * TOK_PAD)], idx_v)
  pltpu.sync_copy(pos_hbm, pos_v)

  bufas = (bufa, bufa2)
  bufbs = (bufb, bufb2)
  gsems = (gsem, gsem2)
  ssems = (ssem, ssem2)

  def issue_gathers(p, s):
    ib = p * TOK_PAD
    pltpu.async_copy(table_hbm.at[idx_v.at[pl.ds(ib, CHUNK_A)]],
                     bufas[s], gsems[s])
    pltpu.async_copy(table_hbm.at[idx_v.at[pl.ds(ib + CHUNK_A, CHUNK_B)]],
                     bufbs[s], gsems[s])

  def wait_gathers(s):
    pltpu.make_async_copy(table_hbm.at[idx_v.at[pl.ds(0, CHUNK_A)]],
                          bufas[s], gsems[s]).wait()
    pltpu.make_async_copy(table_hbm.at[idx_v.at[pl.ds(0, CHUNK_B)]],
                          bufbs[s], gsems[s]).wait()

  def wait_scatters(s):
    pltpu.make_async_copy(bufas[s], out_hbm.at[pl.ds(0, CHUNK_A)],
                          ssems[s]).wait()
    pltpu.make_async_copy(bufbs[s], out_hbm.at[pl.ds(0, CHUNK_B)],
                          ssems[s]).wait()

  issue_gathers(0, 0)
  for p in range(B_PER_W):
    s = p % 2
    wait_gathers(s)
    if p + 1 < B_PER_W:
      issue_gathers(p + 1, 1 - s)
    if p >= 2:
      wait_scatters(s)

    def row_a(r, c2):
      for c in range(C_PER_ROW):
        sl = pl.ds(c * LANES, LANES)
        bufas[s][r, sl] = bufas[s][r, sl] + pos_v[r, sl]
      return c2

    def row_b(r, c2):
      for c in range(C_PER_ROW):
        sl = pl.ds(c * LANES, LANES)
        bufbs[s][r, sl] = bufbs[s][r, sl] + pos_v[CHUNK_A + r, sl]
      return c2

    lax.fori_loop(0, CHUNK_A, row_a, 0)
    lax.fori_loop(0, CHUNK_B, row_b, 0)
    ob = p * NUM_TOKEN
    pltpu.async_copy(bufas[s], out_hbm.at[pl.ds(base + ob, CHUNK_A)],
                     ssems[s])
    pltpu.async_copy(bufbs[s], out_hbm.at[pl.ds(base + ob + CHUNK_A,
                                                CHUNK_B)], ssems[s])
  wait_scatters(0)
  wait_scatters(1)


@jax.jit
def _emb(tok1, table, positionembed):
  mesh = plsc.VectorSubcoreMesh(core_axis_name="c", subcore_axis_name="s")
  run = functools.partial(
      pl.kernel,
      mesh=mesh,
      compiler_params=pltpu.CompilerParams(use_tc_tiling_on_sc=False,
                                           needs_layout_passes=False),
      out_type=jax.ShapeDtypeStruct((B_TOTAL, NUM_EMBED), jnp.float32),
      scratch_types=[
          pltpu.VMEM((B_PER_W * TOK_PAD,), jnp.int32),
          pltpu.VMEM((NUM_TOKEN, NUM_EMBED), jnp.float32),
          pltpu.VMEM((CHUNK_A, NUM_EMBED), jnp.float32),
          pltpu.VMEM((CHUNK_B, NUM_EMBED), jnp.float32),
          pltpu.VMEM((CHUNK_A, NUM_EMBED), jnp.float32),
          pltpu.VMEM((CHUNK_B, NUM_EMBED), jnp.float32),
          pltpu.SemaphoreType.DMA,
          pltpu.SemaphoreType.DMA,
          pltpu.SemaphoreType.DMA,
          pltpu.SemaphoreType.DMA,
      ],
  )(_emb_kernel)
  return run(tok1, table, positionembed)


def kernel(tokens, table, positionembed):
  tokens_p = jnp.pad(tokens.astype(jnp.int32),
                     ((0, 0), (0, TOK_PAD - NUM_TOKEN)))
  tok1 = _tok_split(tokens_p)
  out = _emb(tok1, table, positionembed)
  return out.reshape(BATCH, NUM_TOKEN, NUM_EMBED)
